# baseline (speedup 1.0000x reference)
import jax
import jax.numpy as jnp
from jax.experimental import pallas as pl
from jax.experimental.pallas import tpu as pltpu

_B = 16
_DW = 384
_DH = 384
_C = 3
_MH = 512
_MW = 512
_PH = _MH - _DW
_PW = _MW - _DH


def _pad_kernel(d_hbm, o_hbm, nan_vmem, sem):
    data_copies = [
        pltpu.make_async_copy(
            d_hbm.at[b],
            o_hbm.at[b, pl.ds(0, _DW), pl.ds(0, _DH), :],
            sem,
        )
        for b in range(_B)
    ]
    for c in data_copies:
        c.start()

    nan_vmem[...] = jnp.full((_PH, _MW, _C), jnp.nan, jnp.float32)

    nan_copies = []
    for b in range(_B):
        nan_copies.append(
            pltpu.make_async_copy(
                nan_vmem, o_hbm.at[b, pl.ds(_DW, _PH), :, :], sem
            )
        )
        for i in range(_DW // _PH):
            nan_copies.append(
                pltpu.make_async_copy(
                    nan_vmem.at[:, pl.ds(0, _PW), :],
                    o_hbm.at[b, pl.ds(_PH * i, _PH), pl.ds(_DH, _PW), :],
                    sem,
                )
            )
    for c in nan_copies:
        c.start()
    for c in data_copies:
        c.wait()
    for c in nan_copies:
        c.wait()


def kernel(data, shape):
    return pl.pallas_call(
        _pad_kernel,
        in_specs=[pl.BlockSpec(memory_space=pl.ANY)],
        out_specs=pl.BlockSpec(memory_space=pl.ANY),
        out_shape=jax.ShapeDtypeStruct((_B, _MH, _MW, _C), jnp.float32),
        scratch_shapes=[
            pltpu.VMEM((_PH, _MW, _C), jnp.float32),
            pltpu.SemaphoreType.DMA,
        ],
    )(data)


# speedup vs baseline: 879.0720x; 879.0720x over previous
import jax
import jax.numpy as jnp
from jax.experimental import pallas as pl

_B = 16
_C = 3
_D = 384
_M = 512
_P = _M - _D
_N = _B * _C


def _pad_kernel(d_ref, o_ref):
    o_ref[0, : _D, : _D] = d_ref[0]
    o_ref[0, : _D, _D :] = jnp.full((_D, _P), jnp.nan, jnp.float32)
    o_ref[0, _D :, :] = jnp.full((_P, _M), jnp.nan, jnp.float32)


def kernel(data, shape):
    planes = jnp.transpose(data, (0, 3, 1, 2)).reshape(_N, _D, _D)
    out = pl.pallas_call(
        _pad_kernel,
        grid=(_N,),
        in_specs=[pl.BlockSpec((1, _D, _D), lambda i: (i, 0, 0))],
        out_specs=pl.BlockSpec((1, _M, _M), lambda i: (i, 0, 0)),
        out_shape=jax.ShapeDtypeStruct((_N, _M, _M), jnp.float32),
    )(planes)
    return jnp.transpose(out.reshape(_B, _C, _M, _M), (0, 2, 3, 1))
